# baseline (speedup 1.0000x reference)
import functools

import jax
import jax.numpy as jnp
from jax import lax
from jax.experimental import pallas as pl
from jax.experimental.pallas import tpu as pltpu
from jax.experimental.pallas import tpu_sc as plsc

V = 50000
N = 4
VP = 50176
SLICE = VP // 16
E = 3 * 100000
EP = 307200
TILE_E = EP // 16
CHUNK = 128
NCHUNK = TILE_E // CHUNK


def _decode(kv):
    shift = jnp.full((16,), 16, dtype=jnp.int32)
    e0 = lax.shift_right_logical(kv, shift)
    e1 = jnp.bitwise_and(kv, jnp.int32(0xFFFF))
    return e0, e1


def _make_sc_body(correction):
    def body(keys_h, prev_h, vt_h, s_h, deg_h,
             plane_v, keys_v, prev_v, zero_v,
             idx0_a, idx1_a, val0_a, val1_a,
             idx0_b, idx1_b, val0_b, val1_b,
             out_v, flags_s, acc_sh, sem_a, sem_b):
        c = lax.axis_index("c")
        s = lax.axis_index("s")
        ebase = s * TILE_E

        pltpu.sync_copy(keys_h.at[pl.ds(ebase, TILE_E)], keys_v)
        if correction:
            pltpu.sync_copy(prev_h.at[pl.ds(ebase, TILE_E)], prev_v)

        def _zbody(i, carry):
            zero_v[pl.ds(i * 16, 16)] = jnp.zeros((16,), jnp.float32)
            return carry
        lax.fori_loop(0, SLICE // 16, _zbody, 0)

        if correction:
            def _fbody(k, carry):
                base = k * CHUNK
                flag = jnp.int32(0)
                for j in range(CHUNK // 16):
                    kv = keys_v[pl.ds(base + j * 16, 16)]
                    pv = prev_v[pl.ds(base + j * 16, 16)]
                    eq = jnp.where(kv == pv, jnp.full((16,), 1, jnp.int32),
                                   jnp.zeros((16,), jnp.int32))
                    flag = flag + jnp.max(eq)
                flags_s[k] = flag
                return carry + flag
            tile_any = lax.fori_loop(0, NCHUNK, _fbody, jnp.int32(0))

        buf_a = (idx0_a, idx1_a, val0_a, val1_a, sem_a)
        buf_b = (idx0_b, idx1_b, val0_b, val1_b, sem_b)

        def _compute_chunk(k, buf, gather_plane):
            idx0_v, idx1_v, val0_v, val1_v, _ = buf
            base = k * CHUNK
            for j in range(CHUNK // 16):
                kv = keys_v[pl.ds(base + j * 16, 16)]
                e0, e1 = _decode(kv)
                if correction:
                    pv = prev_v[pl.ds(base + j * 16, 16)]
                    m = kv == pv
                    zero16 = jnp.zeros((16,), jnp.float32)
                    if gather_plane:
                        v0 = jnp.where(m, plsc.load_gather(plane_v, [e1]), zero16)
                        v1 = jnp.where(m, plsc.load_gather(plane_v, [e0]), zero16)
                    else:
                        mv = jnp.where(m, jnp.full((16,), 1.0, jnp.float32), zero16)
                        v0 = mv
                        v1 = mv
                else:
                    if gather_plane:
                        v0 = plsc.load_gather(plane_v, [e1])
                        v1 = plsc.load_gather(plane_v, [e0])
                    else:
                        v0 = jnp.full((16,), 1.0, jnp.float32)
                        v1 = v0
                idx0_v[pl.ds(j * 16, 16)] = e0
                val0_v[pl.ds(j * 16, 16)] = v0
                idx1_v[pl.ds(j * 16, 16)] = e1
                val1_v[pl.ds(j * 16, 16)] = v1

        def _fire(buf):
            idx0_v, idx1_v, val0_v, val1_v, sem = buf
            pltpu.async_copy(val0_v, acc_sh.at[idx0_v], sem, add=True)
            pltpu.async_copy(val1_v, acc_sh.at[idx1_v], sem, add=True)

        def _drain(buf):
            idx0_v, idx1_v, val0_v, val1_v, sem = buf
            pltpu.make_async_copy(val0_v, acc_sh.at[idx0_v], sem).wait()
            pltpu.make_async_copy(val1_v, acc_sh.at[idx1_v], sem).wait()

        def _scatter_pass(gather_plane):
            if correction:
                def _chunk(k, carry):
                    @pl.when(flags_s[k] > 0)
                    def _():
                        _compute_chunk(k, buf_a, gather_plane)
                        _fire(buf_a)
                        _drain(buf_a)
                    return carry
                lax.fori_loop(0, NCHUNK, _chunk, 0)
            else:
                _compute_chunk(0, buf_a, gather_plane)
                _fire(buf_a)
                _compute_chunk(1, buf_b, gather_plane)
                _fire(buf_b)

                def _pair(i, carry):
                    _drain(buf_a)
                    _compute_chunk(2 * i + 2, buf_a, gather_plane)
                    _fire(buf_a)
                    _drain(buf_b)
                    _compute_chunk(2 * i + 3, buf_b, gather_plane)
                    _fire(buf_b)
                    return carry
                lax.fori_loop(0, (NCHUNK - 2) // 2, _pair, 0)
                _drain(buf_a)
                _drain(buf_b)

        for p in range(6):
            plane_idx = 6 * c + p
            if correction:
                @pl.when(tile_any > 0)
                def _stage():
                    pltpu.sync_copy(vt_h.at[pl.ds(plane_idx * VP, VP)], plane_v)
            else:
                pltpu.sync_copy(vt_h.at[pl.ds(plane_idx * VP, VP)], plane_v)
            pltpu.sync_copy(zero_v, acc_sh.at[pl.ds(s * SLICE, SLICE)])
            plsc.subcore_barrier()
            if correction:
                @pl.when(tile_any > 0)
                def _work():
                    _scatter_pass(gather_plane=True)
            else:
                _scatter_pass(gather_plane=True)
            plsc.subcore_barrier()
            pltpu.sync_copy(acc_sh.at[pl.ds(s * SLICE, SLICE)], out_v)
            pltpu.sync_copy(out_v, s_h.at[pl.ds(plane_idx * VP + s * SLICE, SLICE)])

        @pl.when(c == 0)
        def _deg_pass():
            pltpu.sync_copy(zero_v, acc_sh.at[pl.ds(s * SLICE, SLICE)])
            plsc.subcore_barrier()
            if correction:
                @pl.when(tile_any > 0)
                def _workd():
                    _scatter_pass(gather_plane=False)
            else:
                _scatter_pass(gather_plane=False)
            plsc.subcore_barrier()
            pltpu.sync_copy(acc_sh.at[pl.ds(s * SLICE, SLICE)], out_v)
            pltpu.sync_copy(out_v, deg_h.at[pl.ds(s * SLICE, SLICE)])

    return body


def _make_sc_kernel(correction):
    return functools.partial(
        pl.kernel,
        out_type=(
            jax.ShapeDtypeStruct((N * 3 * VP,), jnp.float32),
            jax.ShapeDtypeStruct((VP,), jnp.float32),
        ),
        mesh=plsc.VectorSubcoreMesh(core_axis_name="c", subcore_axis_name="s"),
        scratch_types=(
            pltpu.VMEM((VP,), jnp.float32),
            pltpu.VMEM((TILE_E,), jnp.int32),
            pltpu.VMEM((TILE_E,), jnp.int32),
            pltpu.VMEM((SLICE,), jnp.float32),
            pltpu.VMEM((CHUNK,), jnp.int32),
            pltpu.VMEM((CHUNK,), jnp.int32),
            pltpu.VMEM((CHUNK,), jnp.float32),
            pltpu.VMEM((CHUNK,), jnp.float32),
            pltpu.VMEM((CHUNK,), jnp.int32),
            pltpu.VMEM((CHUNK,), jnp.int32),
            pltpu.VMEM((CHUNK,), jnp.float32),
            pltpu.VMEM((CHUNK,), jnp.float32),
            pltpu.VMEM((SLICE,), jnp.float32),
            pltpu.SMEM((NCHUNK,), jnp.int32),
            pltpu.VMEM_SHARED((VP,), jnp.float32),
            pltpu.SemaphoreType.DMA,
            pltpu.SemaphoreType.DMA,
        ),
        compiler_params=pltpu.CompilerParams(needs_layout_passes=False),
    )(_make_sc_body(correction))


_sc_raw = _make_sc_kernel(correction=False)
_sc_cor = _make_sc_kernel(correction=True)


def _tc_finish(sr_ref, sc_ref, dr_ref, dc_ref, vt_ref, out_ref):
    S = sr_ref[...] - sc_ref[...]
    vt = vt_ref[...]
    d = dr_ref[...] - dc_ref[...]
    inv = jnp.where(d > 0.0, 1.0 / jnp.where(d > 0.0, d, 1.0), 0.0)
    R = S * inv - vt
    sq = jnp.sum(R * R, axis=1)
    lane = lax.broadcasted_iota(jnp.int32, (N, VP), 1)
    loss = jnp.where(lane < V, jnp.sqrt(sq), 0.0)
    out_ref[0, 0] = jnp.sum(loss) * (1.0 / (V * N))


def kernel(vertices, faces):
    f = faces.astype(jnp.int32)
    x = jnp.concatenate([f[:, 0], f[:, 1], f[:, 2]])
    y = jnp.concatenate([f[:, 1], f[:, 2], f[:, 0]])
    a = jnp.minimum(x, y).astype(jnp.uint32)
    b = jnp.maximum(x, y).astype(jnp.uint32)
    keys = (a << 16) | b

    pad_i = jnp.arange(EP - E, dtype=jnp.uint32)
    pad_row = jnp.uint32(V) + (pad_i % jnp.uint32(VP - V))
    rawpad = (pad_row << 16) | pad_row
    keys_raw = lax.bitcast_convert_type(jnp.concatenate([keys, rawpad]),
                                        jnp.int32)

    sk = jnp.sort(keys)
    prev = jnp.concatenate([sk[:1] ^ jnp.uint32(1), sk[:-1]])
    padk = (pad_i << 16) | pad_i
    keys_sorted = lax.bitcast_convert_type(jnp.concatenate([sk, padk]),
                                           jnp.int32)
    prev_sorted = lax.bitcast_convert_type(
        jnp.concatenate([prev, padk ^ jnp.uint32(1)]), jnp.int32)

    vt = jnp.pad(jnp.transpose(vertices, (0, 2, 1)),
                 ((0, 0), (0, 0), (0, VP - V)))
    vt_flat = vt.reshape(N * 3 * VP)

    S_raw, deg_raw = _sc_raw(keys_raw, keys_raw, vt_flat)
    S_cor, deg_cor = _sc_cor(keys_sorted, prev_sorted, vt_flat)

    total = pl.pallas_call(
        _tc_finish,
        out_shape=jax.ShapeDtypeStruct((1, 1), jnp.float32),
        out_specs=pl.BlockSpec(memory_space=pltpu.SMEM),
    )(S_raw.reshape(N, 3, VP), S_cor.reshape(N, 3, VP),
      deg_raw.reshape(1, 1, VP), deg_cor.reshape(1, 1, VP), vt)
    return total[0, 0]


# speedup vs baseline: 1.9254x; 1.9254x over previous
import functools

import jax
import jax.numpy as jnp
from jax import lax
from jax.experimental import pallas as pl
from jax.experimental.pallas import tpu as pltpu
from jax.experimental.pallas import tpu_sc as plsc

V = 50000
N = 4
VP = 50176
SLICE = VP // 16
E = 3 * 100000
EP = 307200
TILE_E = EP // 16
CHUNK = 128
NCHUNK = TILE_E // CHUNK


def _decode(kv):
    shift = jnp.full((16,), 16, dtype=jnp.int32)
    e0 = lax.shift_right_logical(kv, shift)
    e1 = jnp.bitwise_and(kv, jnp.int32(0xFFFF))
    return e0, e1


def _sc_body(keys_h, vt_h, s_h, deg_h,
             plane_a, plane_b, keys_v,
             idx0_a, idx1_a, v0a_a, v1a_a, v0b_a, v1b_a,
             idx0_b, idx1_b, v0a_b, v1a_b, v0b_b, v1b_b,
             scr_v, acc_a, acc_b, sem_a, sem_b):
    c = lax.axis_index("c")
    s = lax.axis_index("s")
    ebase = s * TILE_E

    pltpu.sync_copy(keys_h.at[pl.ds(ebase, TILE_E)], keys_v)

    def _fill_zeros():
        def _zbody(i, carry):
            scr_v[pl.ds(i * 16, 16)] = jnp.zeros((16,), jnp.float32)
            return carry
        lax.fori_loop(0, SLICE // 16, _zbody, 0)

    buf_a = (idx0_a, idx1_a, v0a_a, v1a_a, v0b_a, v1b_a, sem_a)
    buf_b = (idx0_b, idx1_b, v0a_b, v1a_b, v0b_b, v1b_b, sem_b)

    def _compute_chunk(k, buf):
        idx0_v, idx1_v, v0a, v1a, v0b, v1b = buf[:6]
        base = k * CHUNK
        for j in range(CHUNK // 16):
            kv = keys_v[pl.ds(base + j * 16, 16)]
            e0, e1 = _decode(kv)
            v0a[pl.ds(j * 16, 16)] = plsc.load_gather(plane_a, [e1])
            v1a[pl.ds(j * 16, 16)] = plsc.load_gather(plane_a, [e0])
            v0b[pl.ds(j * 16, 16)] = plsc.load_gather(plane_b, [e1])
            v1b[pl.ds(j * 16, 16)] = plsc.load_gather(plane_b, [e0])
            idx0_v[pl.ds(j * 16, 16)] = e0
            idx1_v[pl.ds(j * 16, 16)] = e1

    def _fire(buf):
        idx0_v, idx1_v, v0a, v1a, v0b, v1b, sem = buf
        pltpu.async_copy(v0a, acc_a.at[idx0_v], sem, add=True)
        pltpu.async_copy(v1a, acc_a.at[idx1_v], sem, add=True)
        pltpu.async_copy(v0b, acc_b.at[idx0_v], sem, add=True)
        pltpu.async_copy(v1b, acc_b.at[idx1_v], sem, add=True)

    def _drain(buf):
        idx0_v, idx1_v, v0a, v1a, v0b, v1b, sem = buf
        pltpu.make_async_copy(v0a, acc_a.at[idx0_v], sem).wait()
        pltpu.make_async_copy(v1a, acc_a.at[idx1_v], sem).wait()
        pltpu.make_async_copy(v0b, acc_b.at[idx0_v], sem).wait()
        pltpu.make_async_copy(v1b, acc_b.at[idx1_v], sem).wait()

    def _scatter_pass():
        _compute_chunk(0, buf_a)
        _fire(buf_a)
        _compute_chunk(1, buf_b)
        _fire(buf_b)

        def _pair(i, carry):
            _drain(buf_a)
            _compute_chunk(2 * i + 2, buf_a)
            _fire(buf_a)
            _drain(buf_b)
            _compute_chunk(2 * i + 3, buf_b)
            _fire(buf_b)
            return carry
        lax.fori_loop(0, (NCHUNK - 2) // 2, _pair, 0)
        _drain(buf_a)
        _drain(buf_b)

    for q in range(3):
        pa = 6 * c + 2 * q
        pb = pa + 1
        pltpu.sync_copy(vt_h.at[pl.ds(pa * VP, VP)], plane_a)
        pltpu.sync_copy(vt_h.at[pl.ds(pb * VP, VP)], plane_b)
        _fill_zeros()
        pltpu.sync_copy(scr_v, acc_a.at[pl.ds(s * SLICE, SLICE)])
        pltpu.sync_copy(scr_v, acc_b.at[pl.ds(s * SLICE, SLICE)])
        plsc.subcore_barrier()
        _scatter_pass()
        plsc.subcore_barrier()
        pltpu.sync_copy(acc_a.at[pl.ds(s * SLICE, SLICE)], scr_v)
        pltpu.sync_copy(scr_v, s_h.at[pl.ds(pa * VP + s * SLICE, SLICE)])
        pltpu.sync_copy(acc_b.at[pl.ds(s * SLICE, SLICE)], scr_v)
        pltpu.sync_copy(scr_v, s_h.at[pl.ds(pb * VP + s * SLICE, SLICE)])

    def _fill_ones(i, carry):
        v0a_a[pl.ds(i * 16, 16)] = jnp.full((16,), 1.0, jnp.float32)
        v1a_a[pl.ds(i * 16, 16)] = jnp.full((16,), 1.0, jnp.float32)
        return carry
    lax.fori_loop(0, CHUNK // 16, _fill_ones, 0)
    _fill_zeros()
    pltpu.sync_copy(scr_v, acc_a.at[pl.ds(s * SLICE, SLICE)])
    plsc.subcore_barrier()

    def _deg_chunk(k, carry):
        base = k * CHUNK
        for j in range(CHUNK // 16):
            kv = keys_v[pl.ds(base + j * 16, 16)]
            e0, e1 = _decode(kv)
            idx0_a[pl.ds(j * 16, 16)] = e0
            idx1_a[pl.ds(j * 16, 16)] = e1
        pltpu.async_copy(v0a_a, acc_a.at[idx0_a], sem_a, add=True)
        pltpu.async_copy(v1a_a, acc_a.at[idx1_a], sem_a, add=True)
        pltpu.make_async_copy(v0a_a, acc_a.at[idx0_a], sem_a).wait()
        pltpu.make_async_copy(v1a_a, acc_a.at[idx1_a], sem_a).wait()
        return carry
    half = NCHUNK // 2
    lax.fori_loop(c * half, c * half + half, _deg_chunk, 0)
    plsc.subcore_barrier()
    pltpu.sync_copy(acc_a.at[pl.ds(s * SLICE, SLICE)], scr_v)
    pltpu.sync_copy(scr_v, deg_h.at[pl.ds(c * VP + s * SLICE, SLICE)])


_sc_kernel = functools.partial(
    pl.kernel,
    out_type=(
        jax.ShapeDtypeStruct((N * 3 * VP,), jnp.float32),
        jax.ShapeDtypeStruct((2 * VP,), jnp.float32),
    ),
    mesh=plsc.VectorSubcoreMesh(core_axis_name="c", subcore_axis_name="s"),
    scratch_types=(
        pltpu.VMEM((VP,), jnp.float32),
        pltpu.VMEM((VP,), jnp.float32),
        pltpu.VMEM((TILE_E,), jnp.int32),
        pltpu.VMEM((CHUNK,), jnp.int32),
        pltpu.VMEM((CHUNK,), jnp.int32),
        pltpu.VMEM((CHUNK,), jnp.float32),
        pltpu.VMEM((CHUNK,), jnp.float32),
        pltpu.VMEM((CHUNK,), jnp.float32),
        pltpu.VMEM((CHUNK,), jnp.float32),
        pltpu.VMEM((CHUNK,), jnp.int32),
        pltpu.VMEM((CHUNK,), jnp.int32),
        pltpu.VMEM((CHUNK,), jnp.float32),
        pltpu.VMEM((CHUNK,), jnp.float32),
        pltpu.VMEM((CHUNK,), jnp.float32),
        pltpu.VMEM((CHUNK,), jnp.float32),
        pltpu.VMEM((SLICE,), jnp.float32),
        pltpu.VMEM_SHARED((VP,), jnp.float32),
        pltpu.VMEM_SHARED((VP,), jnp.float32),
        pltpu.SemaphoreType.DMA,
        pltpu.SemaphoreType.DMA,
    ),
    compiler_params=pltpu.CompilerParams(needs_layout_passes=False),
)(_sc_body)


def _tc_finish(s_ref, deg_ref, vt_ref, out_ref):
    S = s_ref[...]
    vt = vt_ref[...]
    dh = deg_ref[...]
    d = dh[0:1] + dh[1:2]
    inv = jnp.where(d > 0.0, 1.0 / jnp.where(d > 0.0, d, 1.0), 0.0)
    R = S * inv - vt
    sq = jnp.sum(R * R, axis=1)
    lane = lax.broadcasted_iota(jnp.int32, (N, VP), 1)
    loss = jnp.where(lane < V, jnp.sqrt(sq), 0.0)
    out_ref[0, 0] = jnp.sum(loss) * (1.0 / (V * N))


def kernel(vertices, faces):
    f = faces.astype(jnp.int32)
    x = jnp.concatenate([f[:, 0], f[:, 1], f[:, 2]])
    y = jnp.concatenate([f[:, 1], f[:, 2], f[:, 0]])
    a = jnp.minimum(x, y).astype(jnp.uint32)
    b = jnp.maximum(x, y).astype(jnp.uint32)
    keys = (a << 16) | b
    (sk,) = lax.sort((keys,), dimension=0, is_stable=False, num_keys=1)
    m = jnp.concatenate([jnp.ones((1,), bool), sk[1:] != sk[:-1]])
    idxs = jnp.arange(EP, dtype=jnp.uint32)
    pad_row = jnp.uint32(V) + (idxs % jnp.uint32(VP - V))
    padk = (pad_row << 16) | pad_row
    sk_full = jnp.concatenate([jnp.where(m, sk, padk[:E]), padk[E:]])
    keys_full = lax.bitcast_convert_type(sk_full, jnp.int32)

    vt = jnp.pad(jnp.transpose(vertices, (0, 2, 1)),
                 ((0, 0), (0, 0), (0, VP - V)))
    vt_flat = vt.reshape(N * 3 * VP)

    S_flat, deg2 = _sc_kernel(keys_full, vt_flat)

    total = pl.pallas_call(
        _tc_finish,
        out_shape=jax.ShapeDtypeStruct((1, 1), jnp.float32),
        out_specs=pl.BlockSpec(memory_space=pltpu.SMEM),
    )(S_flat.reshape(N, 3, VP), deg2.reshape(2, 1, VP), vt)
    return total[0, 0]


# speedup vs baseline: 1.9755x; 1.0260x over previous
import functools

import jax
import jax.numpy as jnp
from jax import lax
from jax.experimental import pallas as pl
from jax.experimental.pallas import tpu as pltpu
from jax.experimental.pallas import tpu_sc as plsc

V = 50000
N = 4
VP = 50176
SLICE = VP // 16
E = 3 * 100000
EP = 307200
TILE_E = EP // 16
CHUNK = 128
NCHUNK = TILE_E // CHUNK
HSLICE = SLICE // 2


def _decode(kv):
    shift = jnp.full((16,), 16, dtype=jnp.int32)
    e0 = lax.shift_right_logical(kv, shift)
    e1 = jnp.bitwise_and(kv, jnp.int32(0xFFFF))
    return e0, e1


def _sc_body(keys_h, vt_h, s_h, deg_h,
             plane_a, plane_b, keys_v,
             idx0_a, idx1_a, v0a_a, v1a_a, v0b_a, v1b_a,
             idx0_b, idx1_b, v0a_b, v1a_b, v0b_b, v1b_b,
             idx0_c, idx1_c, v0a_c, v1a_c, v0b_c, v1b_c,
             scr_v, acc_a, acc_b, sem_a, sem_b, sem_c):
    c = lax.axis_index("c")
    s = lax.axis_index("s")
    ebase = s * TILE_E

    pltpu.sync_copy(keys_h.at[pl.ds(ebase, TILE_E)], keys_v)

    def _fill_zeros():
        def _zbody(i, carry):
            scr_v[pl.ds(i * 16, 16)] = jnp.zeros((16,), jnp.float32)
            return carry
        lax.fori_loop(0, HSLICE // 16, _zbody, 0)

    buf_a = (idx0_a, idx1_a, v0a_a, v1a_a, v0b_a, v1b_a, sem_a)
    buf_b = (idx0_b, idx1_b, v0a_b, v1a_b, v0b_b, v1b_b, sem_b)
    buf_c = (idx0_c, idx1_c, v0a_c, v1a_c, v0b_c, v1b_c, sem_c)

    def _compute_chunk(k, buf):
        idx0_v, idx1_v, v0a, v1a, v0b, v1b = buf[:6]
        base = k * CHUNK
        for j in range(CHUNK // 16):
            kv = keys_v[pl.ds(base + j * 16, 16)]
            e0, e1 = _decode(kv)
            v0a[pl.ds(j * 16, 16)] = plsc.load_gather(plane_a, [e1])
            v1a[pl.ds(j * 16, 16)] = plsc.load_gather(plane_a, [e0])
            v0b[pl.ds(j * 16, 16)] = plsc.load_gather(plane_b, [e1])
            v1b[pl.ds(j * 16, 16)] = plsc.load_gather(plane_b, [e0])
            idx0_v[pl.ds(j * 16, 16)] = e0
            idx1_v[pl.ds(j * 16, 16)] = e1

    def _fire(buf):
        idx0_v, idx1_v, v0a, v1a, v0b, v1b, sem = buf
        pltpu.async_copy(v0a, acc_a.at[idx0_v], sem, add=True)
        pltpu.async_copy(v1a, acc_a.at[idx1_v], sem, add=True)
        pltpu.async_copy(v0b, acc_b.at[idx0_v], sem, add=True)
        pltpu.async_copy(v1b, acc_b.at[idx1_v], sem, add=True)

    def _drain(buf):
        idx0_v, idx1_v, v0a, v1a, v0b, v1b, sem = buf
        pltpu.make_async_copy(v0a, acc_a.at[idx0_v], sem).wait()
        pltpu.make_async_copy(v1a, acc_a.at[idx1_v], sem).wait()
        pltpu.make_async_copy(v0b, acc_b.at[idx0_v], sem).wait()
        pltpu.make_async_copy(v1b, acc_b.at[idx1_v], sem).wait()

    def _scatter_pass():
        _compute_chunk(0, buf_a)
        _fire(buf_a)
        _compute_chunk(1, buf_b)
        _fire(buf_b)
        _compute_chunk(2, buf_c)
        _fire(buf_c)

        def _triple(i, carry):
            _drain(buf_a)
            _compute_chunk(3 * i + 3, buf_a)
            _fire(buf_a)
            _drain(buf_b)
            _compute_chunk(3 * i + 4, buf_b)
            _fire(buf_b)
            _drain(buf_c)
            _compute_chunk(3 * i + 5, buf_c)
            _fire(buf_c)
            return carry
        lax.fori_loop(0, (NCHUNK - 3) // 3, _triple, 0)
        _drain(buf_a)
        _drain(buf_b)
        _drain(buf_c)

    for q in range(3):
        pa = 6 * c + 2 * q
        pb = pa + 1
        pltpu.sync_copy(vt_h.at[pl.ds(pa * VP, VP)], plane_a)
        pltpu.sync_copy(vt_h.at[pl.ds(pb * VP, VP)], plane_b)
        _fill_zeros()
        for h in range(2):
            pltpu.sync_copy(scr_v, acc_a.at[pl.ds(s * SLICE + h * HSLICE, HSLICE)])
            pltpu.sync_copy(scr_v, acc_b.at[pl.ds(s * SLICE + h * HSLICE, HSLICE)])
        plsc.subcore_barrier()
        _scatter_pass()
        plsc.subcore_barrier()
        for h in range(2):
            pltpu.sync_copy(acc_a.at[pl.ds(s * SLICE + h * HSLICE, HSLICE)], scr_v)
            pltpu.sync_copy(scr_v, s_h.at[pl.ds(pa * VP + s * SLICE + h * HSLICE, HSLICE)])
            pltpu.sync_copy(acc_b.at[pl.ds(s * SLICE + h * HSLICE, HSLICE)], scr_v)
            pltpu.sync_copy(scr_v, s_h.at[pl.ds(pb * VP + s * SLICE + h * HSLICE, HSLICE)])

    def _fill_ones(i, carry):
        v0a_a[pl.ds(i * 16, 16)] = jnp.full((16,), 1.0, jnp.float32)
        v1a_a[pl.ds(i * 16, 16)] = jnp.full((16,), 1.0, jnp.float32)
        return carry
    lax.fori_loop(0, CHUNK // 16, _fill_ones, 0)
    _fill_zeros()
    for h in range(2):
        pltpu.sync_copy(scr_v, acc_a.at[pl.ds(s * SLICE + h * HSLICE, HSLICE)])
    plsc.subcore_barrier()

    def _deg_chunk(k, carry):
        base = k * CHUNK
        for j in range(CHUNK // 16):
            kv = keys_v[pl.ds(base + j * 16, 16)]
            e0, e1 = _decode(kv)
            idx0_a[pl.ds(j * 16, 16)] = e0
            idx1_a[pl.ds(j * 16, 16)] = e1
        pltpu.async_copy(v0a_a, acc_a.at[idx0_a], sem_a, add=True)
        pltpu.async_copy(v1a_a, acc_a.at[idx1_a], sem_a, add=True)
        pltpu.make_async_copy(v0a_a, acc_a.at[idx0_a], sem_a).wait()
        pltpu.make_async_copy(v1a_a, acc_a.at[idx1_a], sem_a).wait()
        return carry
    half = NCHUNK // 2
    lax.fori_loop(c * half, c * half + half, _deg_chunk, 0)
    plsc.subcore_barrier()
    for h in range(2):
        pltpu.sync_copy(acc_a.at[pl.ds(s * SLICE + h * HSLICE, HSLICE)], scr_v)
        pltpu.sync_copy(scr_v, deg_h.at[pl.ds(c * VP + s * SLICE + h * HSLICE, HSLICE)])


_sc_kernel = functools.partial(
    pl.kernel,
    out_type=(
        jax.ShapeDtypeStruct((N * 3 * VP,), jnp.float32),
        jax.ShapeDtypeStruct((2 * VP,), jnp.float32),
    ),
    mesh=plsc.VectorSubcoreMesh(core_axis_name="c", subcore_axis_name="s"),
    scratch_types=(
        pltpu.VMEM((VP,), jnp.float32),
        pltpu.VMEM((VP,), jnp.float32),
        pltpu.VMEM((TILE_E,), jnp.int32),
        pltpu.VMEM((CHUNK,), jnp.int32),
        pltpu.VMEM((CHUNK,), jnp.int32),
        pltpu.VMEM((CHUNK,), jnp.float32),
        pltpu.VMEM((CHUNK,), jnp.float32),
        pltpu.VMEM((CHUNK,), jnp.float32),
        pltpu.VMEM((CHUNK,), jnp.float32),
        pltpu.VMEM((CHUNK,), jnp.int32),
        pltpu.VMEM((CHUNK,), jnp.int32),
        pltpu.VMEM((CHUNK,), jnp.float32),
        pltpu.VMEM((CHUNK,), jnp.float32),
        pltpu.VMEM((CHUNK,), jnp.float32),
        pltpu.VMEM((CHUNK,), jnp.float32),
        pltpu.VMEM((CHUNK,), jnp.int32),
        pltpu.VMEM((CHUNK,), jnp.int32),
        pltpu.VMEM((CHUNK,), jnp.float32),
        pltpu.VMEM((CHUNK,), jnp.float32),
        pltpu.VMEM((CHUNK,), jnp.float32),
        pltpu.VMEM((CHUNK,), jnp.float32),
        pltpu.VMEM((HSLICE,), jnp.float32),
        pltpu.VMEM_SHARED((VP,), jnp.float32),
        pltpu.VMEM_SHARED((VP,), jnp.float32),
        pltpu.SemaphoreType.DMA,
        pltpu.SemaphoreType.DMA,
        pltpu.SemaphoreType.DMA,
    ),
    compiler_params=pltpu.CompilerParams(needs_layout_passes=False),
)(_sc_body)


def _tc_finish(s_ref, deg_ref, vt_ref, out_ref):
    S = s_ref[...]
    vt = vt_ref[...]
    dh = deg_ref[...]
    d = dh[0:1] + dh[1:2]
    inv = jnp.where(d > 0.0, 1.0 / jnp.where(d > 0.0, d, 1.0), 0.0)
    R = S * inv - vt
    sq = jnp.sum(R * R, axis=1)
    lane = lax.broadcasted_iota(jnp.int32, (N, VP), 1)
    loss = jnp.where(lane < V, jnp.sqrt(sq), 0.0)
    out_ref[0, 0] = jnp.sum(loss) * (1.0 / (V * N))


def kernel(vertices, faces):
    f = faces.astype(jnp.int32)
    x = jnp.concatenate([f[:, 0], f[:, 1], f[:, 2]])
    y = jnp.concatenate([f[:, 1], f[:, 2], f[:, 0]])
    a = jnp.minimum(x, y).astype(jnp.uint32)
    b = jnp.maximum(x, y).astype(jnp.uint32)
    keys = (a << 16) | b
    (sk,) = lax.sort((keys,), dimension=0, is_stable=False, num_keys=1)
    m = jnp.concatenate([jnp.ones((1,), bool), sk[1:] != sk[:-1]])
    idxs = jnp.arange(EP, dtype=jnp.uint32)
    pad_row = jnp.uint32(V) + (idxs % jnp.uint32(VP - V))
    padk = (pad_row << 16) | pad_row
    sk_full = jnp.concatenate([jnp.where(m, sk, padk[:E]), padk[E:]])
    keys_full = lax.bitcast_convert_type(sk_full, jnp.int32)

    vt = jnp.pad(jnp.transpose(vertices, (0, 2, 1)),
                 ((0, 0), (0, 0), (0, VP - V)))
    vt_flat = vt.reshape(N * 3 * VP)

    S_flat, deg2 = _sc_kernel(keys_full, vt_flat)

    total = pl.pallas_call(
        _tc_finish,
        out_shape=jax.ShapeDtypeStruct((1, 1), jnp.float32),
        out_specs=pl.BlockSpec(memory_space=pltpu.SMEM),
    )(S_flat.reshape(N, 3, VP), deg2.reshape(2, 1, VP), vt)
    return total[0, 0]
